# baseline (speedup 1.0000x reference)
import functools

import jax
import jax.numpy as jnp
from jax import lax
from jax.experimental import pallas as pl
from jax.experimental.pallas import tpu as pltpu
from jax.experimental.pallas import tpu_sc as plsc

_NUM_CORES = 2
_NUM_SUBCORES = 16
_NW = _NUM_CORES * _NUM_SUBCORES

_NV = 256
_DP = 128


@jax.jit
def _build_table(Wt, Wtp):
    D, V = Wt.shape
    n_full = V // _NV
    tail0 = n_full * _NV
    v2 = tail0 + Wtp.shape[0]
    K = -(-n_full // _NW)
    K += K % 2
    mesh = plsc.VectorSubcoreMesh(core_axis_name="c", subcore_axis_name="s")

    @functools.partial(
        pl.kernel,
        out_type=jax.ShapeDtypeStruct((v2, _DP), jnp.float32),
        mesh=mesh,
        compiler_params=pltpu.CompilerParams(
            use_tc_tiling_on_sc=True, needs_layout_passes=False
        ),
        scratch_types=[
            pltpu.VMEM((D, _NV), jnp.float32),
            pltpu.VMEM((D, _NV), jnp.float32),
            pltpu.VMEM((_NV, _DP), jnp.float32),
            pltpu.VMEM((_NV, _DP), jnp.float32),
            pltpu.VMEM(Wtp.shape, jnp.float32),
            pltpu.SemaphoreType.DMA,
            pltpu.SemaphoreType.DMA,
            pltpu.SemaphoreType.DMA,
            pltpu.SemaphoreType.DMA,
        ],
    )
    def k(wt_hbm, wtp_hbm, w2_hbm, in0, in1, out0, out1, tbuf, is0, is1, os0, os1):
        wid = lax.axis_index("s") * _NUM_CORES + lax.axis_index("c")

        def chunk(j):
            return jnp.minimum(wid + _NW * j, n_full - 1)

        def i_start(j, buf, sem):
            pltpu.async_copy(wt_hbm.at[:, pl.ds(chunk(j) * _NV, _NV)], buf, sem)

        def i_wait(buf, sem):
            pltpu.make_async_copy(wt_hbm.at[:, pl.ds(0, _NV)], buf, sem).wait()

        def o_start(j, buf, sem):
            pltpu.async_copy(buf, w2_hbm.at[pl.ds(chunk(j) * _NV, _NV)], sem)

        def o_wait(buf, sem):
            pltpu.make_async_copy(buf, w2_hbm.at[pl.ds(0, _NV)], sem).wait()

        rows16 = [
            c0 + lax.iota(jnp.int32, 16) for c0 in range(0, D, 16)
        ]

        def transpose(inb, outb):
            @pl.loop(0, _NV, unroll=8)
            def _(v):
                colv = jnp.full((16,), v, jnp.int32)
                for i, c0 in enumerate(range(0, D, 16)):
                    val = plsc.load_gather(inb, [rows16[i], colv])
                    outb[v, pl.ds(c0, 16)] = val

        i_start(0, in0, is0)
        P = K // 2

        @pl.loop(0, P)
        def _(g):
            j0 = 2 * g
            i_wait(in0, is0)
            i_start(j0 + 1, in1, is1)

            @pl.when(g > 0)
            def _():
                o_wait(out0, os0)

            transpose(in0, out0)
            o_start(j0, out0, os0)

            i_wait(in1, is1)

            @pl.when(g < P - 1)
            def _():
                i_start(j0 + 2, in0, is0)

            @pl.when(g > 0)
            def _():
                o_wait(out1, os1)

            transpose(in1, out1)
            o_start(j0 + 1, out1, os1)

        o_wait(out0, os0)
        o_wait(out1, os1)

        @pl.when(wid == 0)
        def _():
            pltpu.sync_copy(wtp_hbm, tbuf)
            pltpu.sync_copy(tbuf, w2_hbm.at[pl.ds(tail0, Wtp.shape[0])])

    return k(Wt, Wtp)


@functools.partial(jax.jit, static_argnums=(2,))
def _emb_gather(Wp, idx, B):
    b_per_w = B // _NW
    CH = 256
    n_chunks = b_per_w // CH
    assert n_chunks % 2 == 0
    mesh = plsc.VectorSubcoreMesh(core_axis_name="c", subcore_axis_name="s")

    @functools.partial(
        pl.kernel,
        out_type=jax.ShapeDtypeStruct((B, _DP), jnp.float32),
        mesh=mesh,
        compiler_params=pltpu.CompilerParams(use_tc_tiling_on_sc=True),
        scratch_types=[
            pltpu.VMEM((b_per_w,), jnp.int32),
            pltpu.VMEM((CH, _DP), jnp.float32),
            pltpu.VMEM((CH, _DP), jnp.float32),
            pltpu.SemaphoreType.DMA,
            pltpu.SemaphoreType.DMA,
            pltpu.SemaphoreType.DMA,
            pltpu.SemaphoreType.DMA,
        ],
    )
    def k(table_hbm, idx_hbm, out_hbm, idx_v, buf0, buf1, gs0, gs1, ss0, ss1):
        wid = lax.axis_index("s") * _NUM_CORES + lax.axis_index("c")
        base = wid * b_per_w
        pltpu.sync_copy(idx_hbm.at[pl.ds(base, b_per_w)], idx_v)

        def g_start(i, buf, sem):
            pltpu.async_copy(table_hbm.at[idx_v.at[pl.ds(i * CH, CH)]], buf, sem)

        def g_wait(buf, sem):
            pltpu.make_async_copy(
                table_hbm.at[idx_v.at[pl.ds(0, CH)]], buf, sem
            ).wait()

        def s_start(i, buf, sem):
            pltpu.async_copy(buf, out_hbm.at[pl.ds(base + i * CH, CH)], sem)

        def s_wait(buf, sem):
            pltpu.make_async_copy(buf, out_hbm.at[pl.ds(base, CH)], sem).wait()

        g_start(0, buf0, gs0)
        n2 = n_chunks // 2

        @pl.loop(0, n2)
        def _(g):
            i0 = 2 * g

            @pl.when(g > 0)
            def _():
                s_wait(buf1, ss1)

            g_start(i0 + 1, buf1, gs1)
            g_wait(buf0, gs0)
            s_start(i0, buf0, ss0)

            @pl.when(g < n2 - 1)
            def _():
                s_wait(buf0, ss0)
                g_start(i0 + 2, buf0, gs0)

            g_wait(buf1, gs1)
            s_start(i0 + 1, buf1, ss1)

        s_wait(buf0, ss0)
        s_wait(buf1, ss1)

    return k(Wp, idx)


def kernel(x, W):
    B, T = x.shape
    V, D = W.shape
    n_full = V // _NV
    tail0 = n_full * _NV
    tail_rows = V - tail0
    tail_pad = (-tail_rows) % 8
    Wt = W.T
    Wtail = lax.slice(W, (tail0, 0), (V, D))
    Wtp = jnp.pad(Wtail, ((0, tail_pad), (0, _DP - D)))
    W2 = _build_table(Wt, Wtp)
    idx = x.reshape(-1)
    outp = _emb_gather(W2, idx, B * T)
    return outp.reshape(B, T, _DP)[:, :, :D]


# speedup vs baseline: 1.1704x; 1.1704x over previous
import functools

import jax
import jax.numpy as jnp
from jax import lax
from jax.experimental import pallas as pl
from jax.experimental.pallas import tpu as pltpu
from jax.experimental.pallas import tpu_sc as plsc

_NUM_CORES = 2
_NUM_SUBCORES = 16
_NW = _NUM_CORES * _NUM_SUBCORES

_NV = 256
_DP = 128


@jax.jit
def _build_table(Wt, Wtp1d):
    D, V = Wt.shape
    n_full = V // _NV
    tail0 = n_full * _NV
    v2 = tail0 + Wtp1d.shape[0] // _DP
    K = -(-n_full // _NW)
    K += K % 2
    mesh = plsc.VectorSubcoreMesh(core_axis_name="c", subcore_axis_name="s")

    @functools.partial(
        pl.kernel,
        out_type=jax.ShapeDtypeStruct((v2 * _DP,), jnp.float32),
        mesh=mesh,
        compiler_params=pltpu.CompilerParams(
            use_tc_tiling_on_sc=True, needs_layout_passes=False
        ),
        scratch_types=[
            pltpu.VMEM((D, _NV), jnp.float32),
            pltpu.VMEM((D, _NV), jnp.float32),
            pltpu.VMEM((_NV * _DP,), jnp.float32),
            pltpu.VMEM((_NV * _DP,), jnp.float32),
            pltpu.VMEM(Wtp1d.shape, jnp.float32),
            pltpu.SemaphoreType.DMA,
            pltpu.SemaphoreType.DMA,
            pltpu.SemaphoreType.DMA,
            pltpu.SemaphoreType.DMA,
        ],
    )
    def k(wt_hbm, wtp_hbm, w2_hbm, in0, in1, out0, out1, tbuf, is0, is1, os0, os1):
        wid = lax.axis_index("s") * _NUM_CORES + lax.axis_index("c")

        def chunk(j):
            return jnp.minimum(wid + _NW * j, n_full - 1)

        def i_start(j, buf, sem):
            pltpu.async_copy(wt_hbm.at[:, pl.ds(chunk(j) * _NV, _NV)], buf, sem)

        def i_wait(buf, sem):
            pltpu.make_async_copy(wt_hbm.at[:, pl.ds(0, _NV)], buf, sem).wait()

        def o_start(j, buf, sem):
            pltpu.async_copy(
                buf, w2_hbm.at[pl.ds(chunk(j) * (_NV * _DP), _NV * _DP)], sem
            )

        def o_wait(buf, sem):
            pltpu.make_async_copy(
                buf, w2_hbm.at[pl.ds(0, _NV * _DP)], sem
            ).wait()

        lanes = lax.iota(jnp.int32, 16) * _DP
        pv = [lanes + v0 * _DP for v0 in range(0, _NV, 16)]

        def transpose(inb, outb):
            @pl.loop(0, D)
            def _(c):
                for i, v0 in enumerate(range(0, _NV, 16)):
                    val = inb[c, pl.ds(v0, 16)]
                    plsc.store_scatter(outb, [pv[i] + c], val)

        i_start(0, in0, is0)
        P = K // 2

        @pl.loop(0, P)
        def _(g):
            j0 = 2 * g
            i_wait(in0, is0)
            i_start(j0 + 1, in1, is1)

            @pl.when(g > 0)
            def _():
                o_wait(out0, os0)

            transpose(in0, out0)
            o_start(j0, out0, os0)

            i_wait(in1, is1)

            @pl.when(g < P - 1)
            def _():
                i_start(j0 + 2, in0, is0)

            @pl.when(g > 0)
            def _():
                o_wait(out1, os1)

            transpose(in1, out1)
            o_start(j0 + 1, out1, os1)

        o_wait(out0, os0)
        o_wait(out1, os1)

        @pl.when(wid == 0)
        def _():
            pltpu.sync_copy(wtp_hbm, tbuf)
            pltpu.sync_copy(
                tbuf, w2_hbm.at[pl.ds(tail0 * _DP, Wtp1d.shape[0])]
            )

    return k(Wt, Wtp1d)


@functools.partial(jax.jit, static_argnums=(2,))
def _emb_gather(Wp, idx, B):
    b_per_w = B // _NW
    CH = 256
    n_chunks = b_per_w // CH
    assert n_chunks % 2 == 0
    mesh = plsc.VectorSubcoreMesh(core_axis_name="c", subcore_axis_name="s")

    @functools.partial(
        pl.kernel,
        out_type=jax.ShapeDtypeStruct((B, _DP), jnp.float32),
        mesh=mesh,
        compiler_params=pltpu.CompilerParams(use_tc_tiling_on_sc=True),
        scratch_types=[
            pltpu.VMEM((b_per_w,), jnp.int32),
            pltpu.VMEM((CH, _DP), jnp.float32),
            pltpu.VMEM((CH, _DP), jnp.float32),
            pltpu.SemaphoreType.DMA,
            pltpu.SemaphoreType.DMA,
            pltpu.SemaphoreType.DMA,
            pltpu.SemaphoreType.DMA,
        ],
    )
    def k(table_hbm, idx_hbm, out_hbm, idx_v, buf0, buf1, gs0, gs1, ss0, ss1):
        wid = lax.axis_index("s") * _NUM_CORES + lax.axis_index("c")
        base = wid * b_per_w
        pltpu.sync_copy(idx_hbm.at[pl.ds(base, b_per_w)], idx_v)

        def g_start(i, buf, sem):
            pltpu.async_copy(table_hbm.at[idx_v.at[pl.ds(i * CH, CH)]], buf, sem)

        def g_wait(buf, sem):
            pltpu.make_async_copy(
                table_hbm.at[idx_v.at[pl.ds(0, CH)]], buf, sem
            ).wait()

        def s_start(i, buf, sem):
            pltpu.async_copy(buf, out_hbm.at[pl.ds(base + i * CH, CH)], sem)

        def s_wait(buf, sem):
            pltpu.make_async_copy(buf, out_hbm.at[pl.ds(base, CH)], sem).wait()

        g_start(0, buf0, gs0)
        n2 = n_chunks // 2

        @pl.loop(0, n2)
        def _(g):
            i0 = 2 * g

            @pl.when(g > 0)
            def _():
                s_wait(buf1, ss1)

            g_start(i0 + 1, buf1, gs1)
            g_wait(buf0, gs0)
            s_start(i0, buf0, ss0)

            @pl.when(g < n2 - 1)
            def _():
                s_wait(buf0, ss0)
                g_start(i0 + 2, buf0, gs0)

            g_wait(buf1, gs1)
            s_start(i0 + 1, buf1, ss1)

        s_wait(buf0, ss0)
        s_wait(buf1, ss1)

    return k(Wp, idx)


def kernel(x, W):
    B, T = x.shape
    V, D = W.shape
    n_full = V // _NV
    tail0 = n_full * _NV
    tail_rows = V - tail0
    tail_pad = (-tail_rows) % 8
    Wt = W.T
    Wtail = lax.slice(W, (tail0, 0), (V, D))
    Wtp1d = jnp.pad(Wtail, ((0, tail_pad), (0, _DP - D))).reshape(-1)
    W2 = _build_table(Wt, Wtp1d).reshape(-1, _DP)
    idx = x.reshape(-1)
    outp = _emb_gather(W2, idx, B * T)
    return outp.reshape(B, T, _DP)[:, :, :D]
